# Initial kernel scaffold; baseline (speedup 1.0000x reference)
import functools

import jax
import jax.numpy as jnp
from jax import lax
from jax.experimental import pallas as pl
from jax.experimental.pallas import tpu as pltpu
from jax.experimental.pallas import tpu_sc as plsc

VOCAB = 1000000
DIM = 64
B = 16384
NEG = 10
K1 = NEG + 1

_info = plsc.get_sparse_core_info()
NC, NS, LANES = _info.num_cores, _info.num_subcores, _info.num_lanes
NW = NC * NS
BPW = B // NW
CHUNK = 64
NCH = BPW // CHUNK
NGRP = CHUNK // LANES

_mesh = plsc.VectorSubcoreMesh(core_axis_name="c", subcore_axis_name="s")


@functools.partial(
    pl.kernel,
    out_type=jax.ShapeDtypeStruct((K1, B), jnp.float32),
    mesh=_mesh,
    scratch_types=[
        pltpu.VMEM((BPW,), jnp.int32),
        pltpu.VMEM((K1, BPW), jnp.int32),
        pltpu.VMEM((2, CHUNK, DIM), jnp.float32),
        pltpu.VMEM((2, K1, CHUNK, DIM), jnp.float32),
        pltpu.VMEM((K1, BPW), jnp.float32),
        pltpu.SemaphoreType.DMA,
        pltpu.SemaphoreType.DMA,
    ],
)
def _sc_scores(cen_tab, ctx_tab, cen_idx, cn_idx, out,
               cen_idx_v, cn_idx_v, cen_rows, cn_rows, scores_v,
               sem_a, sem_b):
    wid = lax.axis_index("s") * NC + lax.axis_index("c")
    wbase = wid * BPW

    pltpu.sync_copy(cen_idx.at[pl.ds(wbase, BPW)], cen_idx_v)
    for j in range(K1):
        pltpu.sync_copy(cn_idx.at[j, pl.ds(wbase, BPW)], cn_idx_v.at[j])

    def issue(c):
        buf = c % 2
        off = c * CHUNK
        sem = sem_a if buf == 0 else sem_b
        hs = [pltpu.async_copy(
            cen_tab.at[cen_idx_v.at[pl.ds(off, CHUNK)]],
            cen_rows.at[buf], sem)]
        for j in range(K1):
            hs.append(pltpu.async_copy(
                ctx_tab.at[cn_idx_v.at[j, pl.ds(off, CHUNK)]],
                cn_rows.at[buf, j], sem))
        return hs

    def compute(c):
        buf = c % 2
        for g in range(NGRP):
            bv = lax.iota(jnp.int32, LANES) + (g * LANES)

            def body(d, accs):
                dv = jnp.full((LANES,), d, jnp.int32)
                cc = plsc.load_gather(cen_rows.at[buf], [bv, dv])
                return tuple(
                    accs[j] + cc * plsc.load_gather(cn_rows.at[buf, j],
                                                    [bv, dv])
                    for j in range(K1))

            accs = lax.fori_loop(
                0, DIM, body,
                tuple(jnp.zeros((LANES,), jnp.float32) for _ in range(K1)))
            row = c * CHUNK + g * LANES
            for j in range(K1):
                scores_v[j, pl.ds(row, LANES)] = accs[j]

    pending = issue(0)
    for c in range(NCH):
        nxt = issue(c + 1) if c + 1 < NCH else None
        for h in pending:
            h.wait()
        compute(c)
        pending = nxt

    for j in range(K1):
        pltpu.sync_copy(scores_v.at[j], out.at[j, pl.ds(wbase, BPW)])


def _loss_body(s_ref, o_ref):
    x = jnp.clip(s_ref[...], -10.0, 10.0)
    row = lax.broadcasted_iota(jnp.int32, (K1, B), 0)
    y = jnp.where(row == 0, x, -x)
    ll = -jnp.log1p(jnp.exp(-y))
    o_ref[0, 0] = -jnp.sum(ll) / B


def kernel(center_word, context_word, neg_words, center_table, context_table):
    cn_idx = jnp.concatenate(
        [context_word[None, :], neg_words.T], axis=0).astype(jnp.int32)
    scores = _sc_scores(center_table, context_table,
                        center_word.astype(jnp.int32), cn_idx)
    loss = pl.pallas_call(
        _loss_body,
        out_shape=jax.ShapeDtypeStruct((1, 1), jnp.float32),
    )(scores)
    return loss[0, 0]


# speedup vs baseline: 2.5577x; 2.5577x over previous
import functools

import jax
import jax.numpy as jnp
from jax import lax
from jax.experimental import pallas as pl
from jax.experimental.pallas import tpu as pltpu
from jax.experimental.pallas import tpu_sc as plsc

VOCAB = 1000000
DIM = 64
B = 16384
NEG = 10
K1 = NEG + 1

_info = plsc.get_sparse_core_info()
NC, NS, LANES = _info.num_cores, _info.num_subcores, _info.num_lanes
NW = NC * NS
BPW = B // NW
CHUNK = 64
NCH = BPW // CHUNK
NGRP = CHUNK // LANES

_mesh = plsc.VectorSubcoreMesh(core_axis_name="c", subcore_axis_name="s")


@functools.partial(
    pl.kernel,
    out_type=jax.ShapeDtypeStruct((K1 * B,), jnp.float32),
    mesh=_mesh,
    scratch_types=[
        pltpu.VMEM((BPW,), jnp.int32),
        pltpu.VMEM((K1 * BPW,), jnp.int32),
        pltpu.VMEM((2, CHUNK, DIM), jnp.float32),
        pltpu.VMEM((2, K1, CHUNK, DIM), jnp.float32),
        pltpu.VMEM((K1 * BPW,), jnp.float32),
        pltpu.SemaphoreType.DMA,
        pltpu.SemaphoreType.DMA,
    ],
    compiler_params=pltpu.CompilerParams(
        needs_layout_passes=False, use_tc_tiling_on_sc=False),
)
def _sc_scores(cen_tab, ctx_tab, cen_idx, cn_idx, out,
               cen_idx_v, cn_idx_v, cen_rows, cn_rows, scores_v,
               sem_a, sem_b):
    wid = lax.axis_index("s") * NC + lax.axis_index("c")
    wbase = wid * BPW

    pltpu.sync_copy(cen_idx.at[pl.ds(wbase, BPW)], cen_idx_v)
    for j in range(K1):
        pltpu.sync_copy(cn_idx.at[pl.ds(j * B + wbase, BPW)],
                        cn_idx_v.at[pl.ds(j * BPW, BPW)])

    def issue(c):
        buf = c % 2
        off = c * CHUNK
        sem = sem_a if buf == 0 else sem_b
        hs = [pltpu.async_copy(
            cen_tab.at[cen_idx_v.at[pl.ds(off, CHUNK)]],
            cen_rows.at[buf], sem)]
        for j in range(K1):
            hs.append(pltpu.async_copy(
                ctx_tab.at[cn_idx_v.at[pl.ds(j * BPW + off, CHUNK)]],
                cn_rows.at[buf, j], sem))
        return hs

    def compute(c):
        buf = c % 2
        for g in range(NGRP):
            bv = lax.iota(jnp.int32, LANES) + (g * LANES)

            def body(d, accs):
                dv = jnp.full((LANES,), d, jnp.int32)
                cc = plsc.load_gather(cen_rows.at[buf], [bv, dv])
                return tuple(
                    accs[j] + cc * plsc.load_gather(cn_rows.at[buf, j],
                                                    [bv, dv])
                    for j in range(K1))

            accs = lax.fori_loop(
                0, DIM, body,
                tuple(jnp.zeros((LANES,), jnp.float32) for _ in range(K1)))
            row = c * CHUNK + g * LANES
            for j in range(K1):
                scores_v[pl.ds(j * BPW + row, LANES)] = accs[j]

    pending = issue(0)
    for c in range(NCH):
        nxt = issue(c + 1) if c + 1 < NCH else None
        for h in pending:
            h.wait()
        compute(c)
        pending = nxt

    for j in range(K1):
        pltpu.sync_copy(scores_v.at[pl.ds(j * BPW, BPW)],
                        out.at[pl.ds(j * B + wbase, BPW)])


def _loss_body(s_ref, o_ref):
    x = jnp.clip(s_ref[...], -10.0, 10.0)
    row = lax.broadcasted_iota(jnp.int32, (K1, B), 0)
    y = jnp.where(row == 0, x, -x)
    ll = -jnp.log1p(jnp.exp(-y))
    o_ref[...] = jnp.reshape(-jnp.sum(ll) / B, (1, 1))


def kernel(center_word, context_word, neg_words, center_table, context_table):
    cn_idx = jnp.concatenate(
        [context_word[None, :], neg_words.T], axis=0).astype(jnp.int32)
    scores = _sc_scores(center_table, context_table,
                        center_word.astype(jnp.int32),
                        cn_idx.reshape(K1 * B))
    scores = scores.reshape(K1, B)
    loss = pl.pallas_call(
        _loss_body,
        out_shape=jax.ShapeDtypeStruct((1, 1), jnp.float32),
    )(scores)
    return loss[0, 0]


# speedup vs baseline: 2.5962x; 1.0151x over previous
import functools

import jax
import jax.numpy as jnp
from jax import lax
from jax.experimental import pallas as pl
from jax.experimental.pallas import tpu as pltpu
from jax.experimental.pallas import tpu_sc as plsc

VOCAB = 1000000
DIM = 64
B = 16384
NEG = 10
K1 = NEG + 1

_info = plsc.get_sparse_core_info()
NC, NS, LANES = _info.num_cores, _info.num_subcores, _info.num_lanes
NW = NC * NS
BPW = B // NW
CHUNK = 64
NCH = BPW // CHUNK
NGRP = CHUNK // LANES

_mesh = plsc.VectorSubcoreMesh(core_axis_name="c", subcore_axis_name="s")


@functools.partial(
    pl.kernel,
    out_type=jax.ShapeDtypeStruct((K1 * B,), jnp.float32),
    mesh=_mesh,
    scratch_types=[
        pltpu.VMEM((BPW,), jnp.int32),
        pltpu.VMEM((K1 * BPW,), jnp.int32),
        pltpu.VMEM((2, CHUNK, DIM), jnp.float32),
        pltpu.VMEM((2, K1, CHUNK, DIM), jnp.float32),
        pltpu.VMEM((K1 * BPW,), jnp.float32),
        pltpu.SemaphoreType.DMA,
        pltpu.SemaphoreType.DMA,
    ],
    compiler_params=pltpu.CompilerParams(
        needs_layout_passes=False, use_tc_tiling_on_sc=False),
)
def _sc_scores(cen_tab, ctx_tab, cen_idx, cn_idx, out,
               cen_idx_v, cn_idx_v, cen_rows, cn_rows, scores_v,
               sem_a, sem_b):
    wid = lax.axis_index("s") * NC + lax.axis_index("c")
    wbase = wid * BPW

    pltpu.sync_copy(cen_idx.at[pl.ds(wbase, BPW)], cen_idx_v)
    for j in range(K1):
        pltpu.sync_copy(cn_idx.at[pl.ds(j * B + wbase, BPW)],
                        cn_idx_v.at[pl.ds(j * BPW, BPW)])

    def issue(c):
        buf = c % 2
        off = c * CHUNK
        sem = sem_a if buf == 0 else sem_b
        hs = [pltpu.async_copy(
            cen_tab.at[cen_idx_v.at[pl.ds(off, CHUNK)]],
            cen_rows.at[buf], sem)]
        for j in range(K1):
            hs.append(pltpu.async_copy(
                ctx_tab.at[cn_idx_v.at[pl.ds(j * BPW + off, CHUNK)]],
                cn_rows.at[buf, j], sem))
        return hs

    def compute(c):
        buf = c % 2
        for g in range(NGRP):
            bv = lax.iota(jnp.int32, LANES) + (g * LANES)

            def body(i, accs):
                d0 = i * 4
                for k in range(4):
                    dv = jnp.full((LANES,), d0 + k, jnp.int32)
                    cc = plsc.load_gather(cen_rows.at[buf], [bv, dv])
                    accs = tuple(
                        accs[j] + cc * plsc.load_gather(cn_rows.at[buf, j],
                                                        [bv, dv])
                        for j in range(K1))
                return accs

            accs = lax.fori_loop(
                0, DIM // 4, body,
                tuple(jnp.zeros((LANES,), jnp.float32) for _ in range(K1)))
            row = c * CHUNK + g * LANES
            for j in range(K1):
                scores_v[pl.ds(j * BPW + row, LANES)] = accs[j]

    pending = issue(0)
    for c in range(NCH):
        nxt = issue(c + 1) if c + 1 < NCH else None
        for h in pending:
            h.wait()
        compute(c)
        pending = nxt

    for j in range(K1):
        pltpu.sync_copy(scores_v.at[pl.ds(j * BPW, BPW)],
                        out.at[pl.ds(j * B + wbase, BPW)])


def _loss_body(s_ref, o_ref):
    x = jnp.clip(s_ref[...], -10.0, 10.0)
    row = lax.broadcasted_iota(jnp.int32, (K1, B), 0)
    y = jnp.where(row == 0, x, -x)
    ll = -jnp.log1p(jnp.exp(-y))
    o_ref[...] = jnp.reshape(-jnp.sum(ll) / B, (1, 1))


def kernel(center_word, context_word, neg_words, center_table, context_table):
    cn_idx = jnp.concatenate(
        [context_word[None, :], neg_words.T], axis=0).astype(jnp.int32)
    scores = _sc_scores(center_table, context_table,
                        center_word.astype(jnp.int32),
                        cn_idx.reshape(K1 * B))
    scores = scores.reshape(K1, B)
    loss = pl.pallas_call(
        _loss_body,
        out_shape=jax.ShapeDtypeStruct((1, 1), jnp.float32),
    )(scores)
    return loss[0, 0]
